# Initial kernel scaffold; baseline (speedup 1.0000x reference)
import functools

import jax
import jax.numpy as jnp
from jax import lax
from jax.experimental import pallas as pl
from jax.experimental.pallas import tpu as pltpu
from jax.experimental.pallas import tpu_sc as plsc

N = 10000
T = 12
H = 128
OUT = 32
E = 320000

NC = 2
NS = 16
NW = NC * NS
CH = 128
EPT = 10112
NCHUNK = EPT // CH
EPAD = NW * EPT
NPAD = 10240
ROWS_PER_TILE_Z = NPAD // NS
ROWS_PER_TILE_X = N // NS

BN_GRU = 500
BN_D = 1000


def _gru_body(x_ref, wih0, whh0, bih0, bhh0, wih1, whh1, bih1, bhh1, wg0,
              h_ref, m_ref, h1_scr, h2_scr):
    t = pl.program_id(1)

    @pl.when(t == 0)
    def _():
        h1_scr[...] = jnp.zeros_like(h1_scr)
        h2_scr[...] = jnp.zeros_like(h2_scr)

    xt = x_ref[0]

    def step(xin, h, wih, whh, bih, bhh):
        gi = jnp.dot(xin, wih[...], preferred_element_type=jnp.float32) + bih[...]
        gh = jnp.dot(h, whh[...], preferred_element_type=jnp.float32) + bhh[...]
        r = jax.nn.sigmoid(gi[:, :H] + gh[:, :H])
        z = jax.nn.sigmoid(gi[:, H:2 * H] + gh[:, H:2 * H])
        n = jnp.tanh(gi[:, 2 * H:] + r * gh[:, 2 * H:])
        return (1.0 - z) * n + z * h

    h1 = step(xt, h1_scr[...], wih0, whh0, bih0, bhh0)
    h2 = step(h1, h2_scr[...], wih1, whh1, bih1, bhh1)
    h1_scr[...] = h1
    h2_scr[...] = h2

    @pl.when(t == T - 1)
    def _():
        h_ref[...] = h2
        m_ref[...] = jnp.dot(h2, wg0[...], preferred_element_type=jnp.float32)


def _run_gru_tc(xt, Wih0T, Whh0T, bih0, bhh0, Wih1T, Whh1T, bih1, bhh1, Wg0T):
    nb = pl.cdiv(N, BN_GRU)
    full = lambda i, t: (0, 0)
    return pl.pallas_call(
        _gru_body,
        grid=(nb, T),
        in_specs=[
            pl.BlockSpec((1, BN_GRU, H), lambda i, t: (t, i, 0)),
            pl.BlockSpec((H, 3 * H), full),
            pl.BlockSpec((H, 3 * H), full),
            pl.BlockSpec((1, 3 * H), full),
            pl.BlockSpec((1, 3 * H), full),
            pl.BlockSpec((H, 3 * H), full),
            pl.BlockSpec((H, 3 * H), full),
            pl.BlockSpec((1, 3 * H), full),
            pl.BlockSpec((1, 3 * H), full),
            pl.BlockSpec((H, H), full),
        ],
        out_specs=[
            pl.BlockSpec((BN_GRU, H), lambda i, t: (i, 0)),
            pl.BlockSpec((BN_GRU, H), lambda i, t: (i, 0)),
        ],
        out_shape=[
            jax.ShapeDtypeStruct((N, H), jnp.float32),
            jax.ShapeDtypeStruct((N, H), jnp.float32),
        ],
        scratch_shapes=[
            pltpu.VMEM((BN_GRU, H), jnp.float32),
            pltpu.VMEM((BN_GRU, H), jnp.float32),
        ],
        compiler_params=pltpu.CompilerParams(
            dimension_semantics=("parallel", "arbitrary")),
    )(xt, Wih0T, Whh0T, bih0, bhh0, Wih1T, Whh1T, bih1, bhh1, Wg0T)


def _sc_conv_body(compute_deg, m_hbm, src_hbm, dst_hbm, *rest):
    if compute_deg:
        (p_hbm, deg_hbm, src_v, dst_v, rows_v, zb, ones_v, zline, acc, dacc,
         sem) = rest
    else:
        (p_hbm, src_v, dst_v, rows_v, zb, ones_v, zline, acc, dacc, sem) = rest
    c = lax.axis_index("c")
    s = lax.axis_index("s")
    w = c * NS + s

    zeros16 = jnp.zeros((16,), jnp.float32)
    ones16 = jnp.ones((16,), jnp.float32)
    for r in range(16):
        for cc in range(H // 16):
            zb[r, pl.ds(cc * 16, 16)] = zeros16
    for cc in range(CH // 16):
        ones_v[pl.ds(cc * 16, 16)] = ones16
        zline[pl.ds(cc * 16, 16)] = zeros16

    def zero_acc(i, carry):
        pltpu.sync_copy(zb, acc.at[pl.ds(s * ROWS_PER_TILE_Z + i * 16, 16)])
        return carry
    lax.fori_loop(0, ROWS_PER_TILE_Z // 16, zero_acc, 0)
    if compute_deg:
        def zero_deg(i, carry):
            pltpu.sync_copy(zline,
                            dacc.at[pl.ds(s * ROWS_PER_TILE_Z + i * CH, CH)])
            return carry
        lax.fori_loop(0, ROWS_PER_TILE_Z // CH, zero_deg, 0)
    plsc.subcore_barrier()

    pltpu.sync_copy(src_hbm.at[w], src_v)
    pltpu.sync_copy(dst_hbm.at[w], dst_v)

    def chunk(j, carry):
        idx_s = src_v.at[j]
        idx_d = dst_v.at[j]
        pltpu.async_copy(m_hbm.at[idx_s], rows_v, sem).wait()
        pltpu.sync_copy(rows_v, acc.at[idx_d], add=True)
        if compute_deg:
            pltpu.sync_copy(ones_v, dacc.at[idx_d], add=True)
        return carry
    lax.fori_loop(0, NCHUNK, chunk, 0)
    plsc.subcore_barrier()

    base = s * ROWS_PER_TILE_X
    pltpu.sync_copy(acc.at[pl.ds(base, ROWS_PER_TILE_X)],
                    p_hbm.at[c, pl.ds(base, ROWS_PER_TILE_X)])
    if compute_deg:
        @pl.when(s == 0)
        def _():
            pltpu.sync_copy(dacc.at[pl.ds(0, N)], deg_hbm.at[c])


def _make_sc_conv(compute_deg):
    mesh = plsc.VectorSubcoreMesh(core_axis_name="c", subcore_axis_name="s")
    if compute_deg:
        out_type = (jax.ShapeDtypeStruct((NC, N, H), jnp.float32),
                    jax.ShapeDtypeStruct((NC, N), jnp.float32))
    else:
        out_type = jax.ShapeDtypeStruct((NC, N, H), jnp.float32)
    return pl.kernel(
        functools.partial(_sc_conv_body, compute_deg),
        out_type=out_type,
        mesh=mesh,
        scratch_types=[
            pltpu.VMEM((NCHUNK, CH), jnp.int32),
            pltpu.VMEM((NCHUNK, CH), jnp.int32),
            pltpu.VMEM((CH, H), jnp.float32),
            pltpu.VMEM((16, H), jnp.float32),
            pltpu.VMEM((CH,), jnp.float32),
            pltpu.VMEM((CH,), jnp.float32),
            pltpu.VMEM_SHARED((NPAD, H), jnp.float32),
            pltpu.VMEM_SHARED((NPAD,), jnp.float32),
            pltpu.SemaphoreType.DMA,
        ],
    )


_sc_conv_deg = _make_sc_conv(True)
_sc_conv = _make_sc_conv(False)


def _mid_body(p_ref, degt_ref, bg_ref, wg_ref, m_ref):
    dsum = degt_ref[:, 0:1] + degt_ref[:, 1:2]
    inv = jnp.where(dsum > 0.0, 1.0 / jnp.where(dsum > 0.0, dsum, 1.0), 0.0)
    s = p_ref[0] + p_ref[1]
    out = jax.nn.relu(s * inv + bg_ref[...])
    m_ref[...] = jnp.dot(out, wg_ref[...], preferred_element_type=jnp.float32)


def _run_mid_tc(p, degt, bg0, Wg1T):
    nb = N // BN_D
    full = lambda i: (0, 0)
    return pl.pallas_call(
        _mid_body,
        grid=(nb,),
        in_specs=[
            pl.BlockSpec((NC, BN_D, H), lambda i: (0, i, 0)),
            pl.BlockSpec((BN_D, NC), lambda i: (i, 0)),
            pl.BlockSpec((1, H), full),
            pl.BlockSpec((H, H), full),
        ],
        out_specs=pl.BlockSpec((BN_D, H), lambda i: (i, 0)),
        out_shape=jax.ShapeDtypeStruct((N, H), jnp.float32),
        compiler_params=pltpu.CompilerParams(
            dimension_semantics=("parallel",)),
    )(p, degt, bg0, Wg1T)


def _final_body(q_ref, degt_ref, h_ref, bg_ref, wskip_ref, bskip_ref,
                gamma_ref, beta_ref, wout_ref, bout_ref, y_ref):
    dsum = degt_ref[:, 0:1] + degt_ref[:, 1:2]
    inv = jnp.where(dsum > 0.0, 1.0 / jnp.where(dsum > 0.0, dsum, 1.0), 0.0)
    s = q_ref[0] + q_ref[1]
    out = jax.nn.relu(s * inv + bg_ref[...])
    res = out + jnp.dot(h_ref[...], wskip_ref[...],
                        preferred_element_type=jnp.float32) + bskip_ref[...]
    mu = jnp.mean(res, axis=-1, keepdims=True)
    var = jnp.mean((res - mu) * (res - mu), axis=-1, keepdims=True)
    ln = gamma_ref[...] * (res - mu) / jnp.sqrt(var + 1e-5) + beta_ref[...]
    y_ref[...] = jnp.dot(ln, wout_ref[...],
                         preferred_element_type=jnp.float32) + bout_ref[...]


def _run_final_tc(q, degt, h, bg1, WskipT, bskip, gamma, beta, WoutT, bout):
    nb = N // BN_D
    full = lambda i: (0, 0)
    return pl.pallas_call(
        _final_body,
        grid=(nb,),
        in_specs=[
            pl.BlockSpec((NC, BN_D, H), lambda i: (0, i, 0)),
            pl.BlockSpec((BN_D, NC), lambda i: (i, 0)),
            pl.BlockSpec((BN_D, H), lambda i: (i, 0)),
            pl.BlockSpec((1, H), full),
            pl.BlockSpec((H, H), full),
            pl.BlockSpec((1, H), full),
            pl.BlockSpec((1, H), full),
            pl.BlockSpec((1, H), full),
            pl.BlockSpec((H, OUT), full),
            pl.BlockSpec((1, OUT), full),
        ],
        out_specs=pl.BlockSpec((BN_D, OUT), lambda i: (i, 0)),
        out_shape=jax.ShapeDtypeStruct((N, OUT), jnp.float32),
        compiler_params=pltpu.CompilerParams(
            dimension_semantics=("parallel",)),
    )(q, degt, h, bg1, WskipT, bskip, gamma, beta, WoutT, bout)


def kernel(x, edge_index, Wih0, Whh0, bih0, bhh0, Wih1, Whh1, bih1, bhh1,
           Wg0, bg0, Wg1, bg1, Wskip, bskip, gamma, beta, Wout, bout):
    src = edge_index[0]
    dst = edge_index[1]
    pad = EPAD - E
    srcp = jnp.concatenate([src, jnp.zeros((pad,), jnp.int32)]
                           ).reshape(NW, NCHUNK, CH)
    dstp = jnp.concatenate([dst, jnp.full((pad,), N, jnp.int32)]
                           ).reshape(NW, NCHUNK, CH)

    xt = jnp.transpose(x, (1, 0, 2))

    row = lambda v: v.reshape(1, -1)
    h, m0 = _run_gru_tc(xt, Wih0.T, Whh0.T, row(bih0), row(bhh0),
                        Wih1.T, Whh1.T, row(bih1), row(bhh1), Wg0.T)

    p, degp = _sc_conv_deg(m0, srcp, dstp)
    degt = jnp.transpose(degp)

    m1 = _run_mid_tc(p, degt, row(bg0), Wg1.T)
    q = _sc_conv(m1, srcp, dstp)

    y = _run_final_tc(q, degt, h, row(bg1), Wskip.T, row(bskip),
                      row(gamma), row(beta), Wout.T, row(bout))
    return y


# speedup vs baseline: 5.5703x; 5.5703x over previous
import functools

import jax
import jax.numpy as jnp
from jax import lax
from jax.experimental import pallas as pl
from jax.experimental.pallas import tpu as pltpu
from jax.experimental.pallas import tpu_sc as plsc

N = 10000
T = 12
H = 128
OUT = 32
E = 320000

NC = 2
NS = 16
NW = NC * NS
CH = 128
EPT = 10112
NCHUNK = EPT // CH
EPAD = NW * EPT
NPAD = 10240
ROWS_PER_TILE_Z = NPAD // NS
ROWS_PER_TILE_X = N // NS

BN_GRU = 400
BN_D = 1000


def _gru_body(x_ref, wih0, whh0, bih0, bhh0, wih1, whh1, bih1, bhh1, wg0,
              h_ref, m_ref, h1_scr, h2_scr):
    t = pl.program_id(1)

    @pl.when(t == 0)
    def _():
        h1_scr[...] = jnp.zeros_like(h1_scr)
        h2_scr[...] = jnp.zeros_like(h2_scr)

    xt = x_ref[0]

    def step(xin, h, wih, whh, bih, bhh):
        gi = jnp.dot(xin, wih[...], preferred_element_type=jnp.float32) + bih[...]
        gh = jnp.dot(h, whh[...], preferred_element_type=jnp.float32) + bhh[...]
        r = jax.nn.sigmoid(gi[:, :H] + gh[:, :H])
        z = jax.nn.sigmoid(gi[:, H:2 * H] + gh[:, H:2 * H])
        n = jnp.tanh(gi[:, 2 * H:] + r * gh[:, 2 * H:])
        return (1.0 - z) * n + z * h

    h1 = step(xt, h1_scr[...], wih0, whh0, bih0, bhh0)
    h2 = step(h1, h2_scr[...], wih1, whh1, bih1, bhh1)
    h1_scr[...] = h1
    h2_scr[...] = h2

    @pl.when(t == T - 1)
    def _():
        h_ref[...] = h2
        m_ref[...] = jnp.dot(h2, wg0[...], preferred_element_type=jnp.float32)


def _run_gru_tc(xt, Wih0T, Whh0T, bih0, bhh0, Wih1T, Whh1T, bih1, bhh1, Wg0T):
    nb = pl.cdiv(N, BN_GRU)
    full = lambda i, t: (0, 0)
    return pl.pallas_call(
        _gru_body,
        grid=(nb, T),
        in_specs=[
            pl.BlockSpec((1, BN_GRU, H), lambda i, t: (t, i, 0)),
            pl.BlockSpec((H, 3 * H), full),
            pl.BlockSpec((H, 3 * H), full),
            pl.BlockSpec((1, 3 * H), full),
            pl.BlockSpec((1, 3 * H), full),
            pl.BlockSpec((H, 3 * H), full),
            pl.BlockSpec((H, 3 * H), full),
            pl.BlockSpec((1, 3 * H), full),
            pl.BlockSpec((1, 3 * H), full),
            pl.BlockSpec((H, H), full),
        ],
        out_specs=[
            pl.BlockSpec((BN_GRU, H), lambda i, t: (i, 0)),
            pl.BlockSpec((BN_GRU, H), lambda i, t: (i, 0)),
        ],
        out_shape=[
            jax.ShapeDtypeStruct((N, H), jnp.float32),
            jax.ShapeDtypeStruct((N, H), jnp.float32),
        ],
        scratch_shapes=[
            pltpu.VMEM((BN_GRU, H), jnp.float32),
            pltpu.VMEM((BN_GRU, H), jnp.float32),
        ],
        compiler_params=pltpu.CompilerParams(
            dimension_semantics=("parallel", "arbitrary")),
    )(xt, Wih0T, Whh0T, bih0, bhh0, Wih1T, Whh1T, bih1, bhh1, Wg0T)


def _sc_conv_body(compute_deg, m_hbm, src_hbm, dst_hbm, *rest):
    if compute_deg:
        (p_hbm, deg_hbm, src_v, dst_v, rows_v, zb, ones_v, zline, acc, dacc,
         sem) = rest
    else:
        (p_hbm, src_v, dst_v, rows_v, zb, ones_v, zline, acc, dacc, sem) = rest
    c = lax.axis_index("c")
    s = lax.axis_index("s")
    w = c * NS + s

    zeros16 = jnp.zeros((16,), jnp.float32)
    ones16 = jnp.ones((16,), jnp.float32)
    for r in range(16):
        for cc in range(H // 16):
            zb[r, pl.ds(cc * 16, 16)] = zeros16
    for cc in range(CH // 16):
        ones_v[pl.ds(cc * 16, 16)] = ones16
        zline[pl.ds(cc * 16, 16)] = zeros16

    def zero_acc(i, carry):
        pltpu.sync_copy(zb, acc.at[pl.ds(s * ROWS_PER_TILE_Z + i * 16, 16)])
        return carry
    lax.fori_loop(0, ROWS_PER_TILE_Z // 16, zero_acc, 0)
    if compute_deg:
        def zero_deg(i, carry):
            pltpu.sync_copy(zline,
                            dacc.at[pl.ds(s * ROWS_PER_TILE_Z + i * CH, CH)])
            return carry
        lax.fori_loop(0, ROWS_PER_TILE_Z // CH, zero_deg, 0)
    plsc.subcore_barrier()

    pltpu.sync_copy(src_hbm.at[w], src_v)
    pltpu.sync_copy(dst_hbm.at[w], dst_v)

    def chunk(j, carry):
        idx_s = src_v.at[j]
        idx_d = dst_v.at[j]
        pltpu.async_copy(m_hbm.at[idx_s], rows_v, sem).wait()
        pltpu.sync_copy(rows_v, acc.at[idx_d], add=True)
        if compute_deg:
            pltpu.sync_copy(ones_v, dacc.at[idx_d], add=True)
        return carry
    lax.fori_loop(0, NCHUNK, chunk, 0)
    plsc.subcore_barrier()

    base = s * ROWS_PER_TILE_Z
    pltpu.sync_copy(acc.at[pl.ds(base, ROWS_PER_TILE_Z)],
                    p_hbm.at[c, pl.ds(base, ROWS_PER_TILE_Z)])
    if compute_deg:
        @pl.when(s == 0)
        def _():
            pltpu.sync_copy(dacc, deg_hbm.at[pl.ds(c * NPAD, NPAD)])


@functools.lru_cache(maxsize=None)
def _make_sc_conv(compute_deg):
    mesh = plsc.VectorSubcoreMesh(core_axis_name="c", subcore_axis_name="s",
                                  num_cores=NC, num_subcores=NS)
    if compute_deg:
        out_type = (jax.ShapeDtypeStruct((NC, NPAD, H), jnp.float32),
                    jax.ShapeDtypeStruct((NC * NPAD,), jnp.float32))
    else:
        out_type = jax.ShapeDtypeStruct((NC, NPAD, H), jnp.float32)
    return pl.kernel(
        functools.partial(_sc_conv_body, compute_deg),
        out_type=out_type,
        mesh=mesh,
        scratch_types=[
            pltpu.VMEM((NCHUNK, CH), jnp.int32),
            pltpu.VMEM((NCHUNK, CH), jnp.int32),
            pltpu.VMEM((CH, H), jnp.float32),
            pltpu.VMEM((16, H), jnp.float32),
            pltpu.VMEM((CH,), jnp.float32),
            pltpu.VMEM((CH,), jnp.float32),
            pltpu.VMEM_SHARED((NPAD, H), jnp.float32),
            pltpu.VMEM_SHARED((NPAD,), jnp.float32),
            pltpu.SemaphoreType.DMA,
        ],
    )


def _mid_body(p_ref, degt_ref, bg_ref, wg_ref, m_ref):
    dsum = degt_ref[:, 0:1] + degt_ref[:, 1:2]
    inv = jnp.where(dsum > 0.0, 1.0 / jnp.where(dsum > 0.0, dsum, 1.0), 0.0)
    s = p_ref[0] + p_ref[1]
    out = jax.nn.relu(s * inv + bg_ref[...])
    m_ref[...] = jnp.dot(out, wg_ref[...], preferred_element_type=jnp.float32)


def _run_mid_tc(p, degt, bg0, Wg1T):
    nb = N // BN_D
    full = lambda i: (0, 0)
    return pl.pallas_call(
        _mid_body,
        grid=(nb,),
        in_specs=[
            pl.BlockSpec((NC, BN_D, H), lambda i: (0, i, 0)),
            pl.BlockSpec((BN_D, NC), lambda i: (i, 0)),
            pl.BlockSpec((1, H), full),
            pl.BlockSpec((H, H), full),
        ],
        out_specs=pl.BlockSpec((BN_D, H), lambda i: (i, 0)),
        out_shape=jax.ShapeDtypeStruct((N, H), jnp.float32),
        compiler_params=pltpu.CompilerParams(
            dimension_semantics=("parallel",)),
    )(p, degt, bg0, Wg1T)


def _final_body(q_ref, degt_ref, h_ref, bg_ref, wskip_ref, bskip_ref,
                gamma_ref, beta_ref, wout_ref, bout_ref, y_ref):
    dsum = degt_ref[:, 0:1] + degt_ref[:, 1:2]
    inv = jnp.where(dsum > 0.0, 1.0 / jnp.where(dsum > 0.0, dsum, 1.0), 0.0)
    s = q_ref[0] + q_ref[1]
    out = jax.nn.relu(s * inv + bg_ref[...])
    res = out + jnp.dot(h_ref[...], wskip_ref[...],
                        preferred_element_type=jnp.float32) + bskip_ref[...]
    mu = jnp.mean(res, axis=-1, keepdims=True)
    var = jnp.mean((res - mu) * (res - mu), axis=-1, keepdims=True)
    ln = gamma_ref[...] * (res - mu) / jnp.sqrt(var + 1e-5) + beta_ref[...]
    y_ref[...] = jnp.dot(ln, wout_ref[...],
                         preferred_element_type=jnp.float32) + bout_ref[...]


def _run_final_tc(q, degt, h, bg1, WskipT, bskip, gamma, beta, WoutT, bout):
    nb = N // BN_D
    full = lambda i: (0, 0)
    return pl.pallas_call(
        _final_body,
        grid=(nb,),
        in_specs=[
            pl.BlockSpec((NC, BN_D, H), lambda i: (0, i, 0)),
            pl.BlockSpec((BN_D, NC), lambda i: (i, 0)),
            pl.BlockSpec((BN_D, H), lambda i: (i, 0)),
            pl.BlockSpec((1, H), full),
            pl.BlockSpec((H, H), full),
            pl.BlockSpec((1, H), full),
            pl.BlockSpec((1, H), full),
            pl.BlockSpec((1, H), full),
            pl.BlockSpec((H, OUT), full),
            pl.BlockSpec((1, OUT), full),
        ],
        out_specs=pl.BlockSpec((BN_D, OUT), lambda i: (i, 0)),
        out_shape=jax.ShapeDtypeStruct((N, OUT), jnp.float32),
        compiler_params=pltpu.CompilerParams(
            dimension_semantics=("parallel",)),
    )(q, degt, h, bg1, WskipT, bskip, gamma, beta, WoutT, bout)


def kernel(x, edge_index, Wih0, Whh0, bih0, bhh0, Wih1, Whh1, bih1, bhh1,
           Wg0, bg0, Wg1, bg1, Wskip, bskip, gamma, beta, Wout, bout):
    src = edge_index[0]
    dst = edge_index[1]
    pad = EPAD - E
    srcp = jnp.concatenate([src, jnp.zeros((pad,), jnp.int32)]
                           ).reshape(NW, NCHUNK, CH)
    dstp = jnp.concatenate([dst, jnp.full((pad,), N, jnp.int32)]
                           ).reshape(NW, NCHUNK, CH)

    xt = jnp.transpose(x, (1, 0, 2))

    row = lambda v: v.reshape(1, -1)
    h, m0 = _run_gru_tc(xt, Wih0.T, Whh0.T, row(bih0), row(bhh0),
                        Wih1.T, Whh1.T, row(bih1), row(bhh1), Wg0.T)

    p, deg_flat = _make_sc_conv(True)(m0, srcp, dstp)
    degt = jnp.stack([deg_flat[:N], deg_flat[NPAD:NPAD + N]], axis=1)

    m1 = _run_mid_tc(p, degt, row(bg0), Wg1.T)
    q = _make_sc_conv(False)(m1, srcp, dstp)

    y = _run_final_tc(q, degt, h, row(bg1), Wskip.T, row(bskip),
                      row(gamma), row(beta), Wout.T, row(bout))
    return y
